# baseline (speedup 1.0000x reference)
import dataclasses
import functools

import jax
import jax.numpy as jnp
from jax import lax
from jax.experimental import pallas as pl
from jax.experimental.pallas import tpu as pltpu
from jax.experimental.pallas import tpu_sc as plsc

N = 10000
NPAD = 10240
E = 320000
F = 128

NC = 2
NS = 16
NW = NC * NS
CHUNK = 128
NCHUNK = E // CHUNK
CPW = -(-NCHUNK // NW)
ROWS_PER_SUB = NPAD // NS
E2 = 327680
NCHUNK2 = E2 // CHUNK
CPW2 = NCHUNK2 // NW
NBUF = 5
EPW = E2 // NW
DEGBLK = 1024

_mesh = plsc.VectorSubcoreMesh(core_axis_name="c", subcore_axis_name="s")


_cp = pltpu.CompilerParams()
if "needs_layout_passes" in pltpu.CompilerParams.__dataclass_fields__:
    _cp = dataclasses.replace(_cp, needs_layout_passes=False)


@functools.partial(
    pl.kernel,
    mesh=_mesh,
    out_type=jax.ShapeDtypeStruct((NW, NPAD), jnp.float32),
    compiler_params=_cp,
    scratch_types=[
        pltpu.VMEM((DEGBLK,), jnp.int32),
        pltpu.VMEM((NPAD,), jnp.float32),
    ],
)
def _deg_kernel(dst_hbm, zeros_hbm, out_hbm, idx_v, deg_v):
    cid = lax.axis_index("c")
    sid = lax.axis_index("s")
    w = sid * NC + cid
    pltpu.sync_copy(zeros_hbm, deg_v)
    ones = jnp.ones((16,), jnp.float32)
    base = w * EPW

    def scan_block(off, size):
        pltpu.sync_copy(dst_hbm.at[pl.ds(base + off, size)],
                        idx_v.at[pl.ds(0, size)])

        @pl.loop(0, size, step=16)
        def _(j):
            plsc.addupdate_scatter(deg_v, [idx_v[pl.ds(j, 16)]], ones)

    @pl.loop(0, EPW // DEGBLK)
    def _(k):
        scan_block(k * DEGBLK, DEGBLK)

    if EPW % DEGBLK:
        scan_block((EPW // DEGBLK) * DEGBLK, EPW % DEGBLK)

    pltpu.sync_copy(deg_v, out_hbm.at[w])


IDXBLK = 8
HCHUNK = CHUNK // 2
NBLK = CPW2 // IDXBLK


@functools.partial(
    pl.kernel,
    mesh=_mesh,
    out_type=jax.ShapeDtypeStruct((NC, NPAD, F), jnp.float32),
    scratch_types=[
        pltpu.VMEM((IDXBLK, CHUNK), jnp.int32),
        pltpu.VMEM((IDXBLK, CHUNK), jnp.int32),
        pltpu.VMEM((IDXBLK, CHUNK), jnp.int32),
        pltpu.VMEM((IDXBLK, CHUNK), jnp.int32),
        pltpu.VMEM((CHUNK, F), jnp.float32),
        pltpu.VMEM((CHUNK, F), jnp.float32),
        pltpu.VMEM((HCHUNK,), jnp.int32),
        pltpu.VMEM((HCHUNK,), jnp.int32),
        pltpu.VMEM((HCHUNK,), jnp.int32),
        pltpu.VMEM((HCHUNK,), jnp.int32),
        pltpu.VMEM((CHUNK,), jnp.int32),
        pltpu.VMEM((CHUNK,), jnp.int32),
        pltpu.VMEM_SHARED((NPAD, F), jnp.float32),
        pltpu.SemaphoreType.DMA,
        pltpu.SemaphoreType.DMA,
        pltpu.SemaphoreType.DMA,
        pltpu.SemaphoreType.DMA,
    ],
)
def _agg_kernel(y_hbm, src2_hbm, dst2_hbm, out_hbm,
                si_a, si_b, di_a, di_b, rows0, rows1,
                dh0a, dh0b, dh1a, dh1b, sc0, sc1,
                acc_sh, gs0, gs1, ss_a, ss_b):
    cid = lax.axis_index("c")
    sid = lax.axis_index("s")
    rr0 = sid * ROWS_PER_SUB

    @pl.loop(0, CHUNK)
    def _(r):
        for c in range(0, F, 16):
            rows0[r, pl.ds(c, 16)] = jnp.zeros((16,), jnp.float32)

    @pl.loop(0, ROWS_PER_SUB, step=CHUNK)
    def _(rr):
        pltpu.sync_copy(rows0, acc_sh.at[pl.ds(rr0 + rr, CHUNK)])

    w = sid * NC + cid
    row0 = w * CPW2

    def load_blk(b, si, di):
        pltpu.sync_copy(src2_hbm.at[pl.ds(row0 + b * IDXBLK, IDXBLK)], si)
        pltpu.sync_copy(dst2_hbm.at[pl.ds(row0 + b * IDXBLK, IDXBLK)], di)

    def gather(si, m, rbuf, sem, sc):
        for t in range(0, CHUNK, 16):
            sc[pl.ds(t, 16)] = si[m, pl.ds(t, 16)]
        pltpu.async_copy(y_hbm.at[sc], rbuf, sem)

    def wait_g(rbuf, sem):
        pltpu.make_async_copy(y_hbm.at[pl.ds(0, CHUNK)], rbuf, sem).wait()

    def scatter(di, m, rbuf, dca, dcb):
        for t in range(0, HCHUNK, 16):
            dca[pl.ds(t, 16)] = di[m, pl.ds(t, 16)]
            dcb[pl.ds(t, 16)] = di[m, pl.ds(HCHUNK + t, 16)]
        pltpu.async_copy(rbuf.at[pl.ds(0, HCHUNK)], acc_sh.at[dca], ss_a,
                         add=True)
        pltpu.async_copy(rbuf.at[pl.ds(HCHUNK, HCHUNK)], acc_sh.at[dcb], ss_b,
                         add=True)
        pltpu.make_async_copy(y_hbm.at[pl.ds(0, HCHUNK)],
                              rbuf.at[pl.ds(0, HCHUNK)], ss_a).wait()
        pltpu.make_async_copy(y_hbm.at[pl.ds(0, HCHUNK)],
                              rbuf.at[pl.ds(HCHUNK, HCHUNK)], ss_b).wait()

    def process(si, di, nsi, next_guard):
        @pl.loop(0, (IDXBLK - 2) // 2)
        def _(mm):
            m0 = 2 * mm
            wait_g(rows0, gs0)
            scatter(di, m0, rows0, dh0a, dh0b)
            gather(si, m0 + 2, rows0, gs0, sc0)
            wait_g(rows1, gs1)
            scatter(di, m0 + 1, rows1, dh1a, dh1b)
            gather(si, m0 + 3, rows1, gs1, sc1)

        wait_g(rows0, gs0)
        scatter(di, IDXBLK - 2, rows0, dh0a, dh0b)
        if next_guard is None:
            gather(nsi, 0, rows0, gs0, sc0)
        else:
            @pl.when(next_guard)
            def _():
                gather(nsi, 0, rows0, gs0, sc0)
        wait_g(rows1, gs1)
        scatter(di, IDXBLK - 1, rows1, dh1a, dh1b)
        if next_guard is None:
            gather(nsi, 1, rows1, gs1, sc1)
        else:
            @pl.when(next_guard)
            def _():
                gather(nsi, 1, rows1, gs1, sc1)

    plsc.subcore_barrier()

    load_blk(0, si_a, di_a)
    gather(si_a, 0, rows0, gs0, sc0)
    gather(si_a, 1, rows1, gs1, sc1)

    @pl.loop(0, NBLK // 2)
    def _(j):
        b_a = 2 * j
        load_blk(b_a + 1, si_b, di_b)
        process(si_a, di_a, si_b, None)

        @pl.when(j < NBLK // 2 - 1)
        def _():
            load_blk(b_a + 2, si_a, di_a)

        process(si_b, di_b, si_a, j < NBLK // 2 - 1)

    plsc.subcore_barrier()
    pltpu.sync_copy(acc_sh.at[pl.ds(rr0, ROWS_PER_SUB)],
                    out_hbm.at[cid].at[pl.ds(rr0, ROWS_PER_SUB)])


_MM_BLK = 1024


def _mm_body(x_ref, w_ref, b_ref, h_ref):
    h_ref[...] = (
        jnp.dot(x_ref[...], w_ref[...], preferred_element_type=jnp.float32)
        + b_ref[...]
    )


def _matmul(x, W, b2):
    return pl.pallas_call(
        _mm_body,
        grid=(NPAD // _MM_BLK,),
        in_specs=[
            pl.BlockSpec((_MM_BLK, F), lambda i: (i, 0)),
            pl.BlockSpec((F, F), lambda i: (0, 0)),
            pl.BlockSpec((1, F), lambda i: (0, 0)),
        ],
        out_specs=pl.BlockSpec((_MM_BLK, F), lambda i: (i, 0)),
        out_shape=jax.ShapeDtypeStruct((NPAD, F), jnp.float32),
    )(x, W, b2)


def _dis_from_parts(dp):
    deg = jnp.sum(dp, axis=0)[:, None]
    return lax.rsqrt(jnp.maximum(deg, 1.0))


def _scale_body(h_ref, dp_ref, y_ref):
    y_ref[...] = h_ref[...] * _dis_from_parts(dp_ref[...])


def _scale(h, deg_parts):
    return pl.pallas_call(
        _scale_body,
        grid=(NPAD // _MM_BLK,),
        in_specs=[
            pl.BlockSpec((_MM_BLK, F), lambda i: (i, 0)),
            pl.BlockSpec((NW, _MM_BLK), lambda i: (0, i)),
        ],
        out_specs=pl.BlockSpec((_MM_BLK, F), lambda i: (i, 0)),
        out_shape=jax.ShapeDtypeStruct((NPAD, F), jnp.float32),
    )(h, deg_parts)


def _final_body(acc_ref, dp_ref, o_ref):
    z = (acc_ref[0] + acc_ref[1]) * _dis_from_parts(dp_ref[...])
    m = jnp.max(z, axis=1, keepdims=True)
    lse = jnp.log(jnp.sum(jnp.exp(z - m), axis=1, keepdims=True)) + m
    o_ref[...] = z - lse


def _final(acc, deg_parts):
    return pl.pallas_call(
        _final_body,
        grid=(NPAD // _MM_BLK,),
        in_specs=[
            pl.BlockSpec((NC, _MM_BLK, F), lambda i: (0, i, 0)),
            pl.BlockSpec((NW, _MM_BLK), lambda i: (0, i)),
        ],
        out_specs=pl.BlockSpec((_MM_BLK, F), lambda i: (i, 0)),
        out_shape=jax.ShapeDtypeStruct((NPAD, F), jnp.float32),
    )(acc, deg_parts)


def kernel(inputs, edge_index, epoch, W, b):
    del epoch
    src = edge_index[0].astype(jnp.int32)
    dst = edge_index[1].astype(jnp.int32)
    pad = E2 - E
    pad_iota = jnp.arange(pad, dtype=jnp.int32)
    src_p = jnp.concatenate([src, pad_iota % N])
    dst_p = jnp.concatenate([dst, N + pad_iota % (NPAD - N)])
    src2d = src_p.reshape(NCHUNK2, CHUNK)
    dst2d = dst_p.reshape(NCHUNK2, CHUNK)
    zeros1 = jnp.zeros((NPAD,), jnp.float32)
    b2 = b.reshape(1, F)
    x_pad = jnp.pad(inputs, ((0, NPAD - N), (0, 0)))

    deg_parts = _deg_kernel(dst_p, zeros1)
    h = _matmul(x_pad, W, b2)
    y = _scale(h, deg_parts)
    acc = _agg_kernel(y, src2d, dst2d)
    return _final(acc, deg_parts)[:N]


# speedup vs baseline: 1.0420x; 1.0420x over previous
import dataclasses
import functools

import jax
import jax.numpy as jnp
from jax import lax
from jax.experimental import pallas as pl
from jax.experimental.pallas import tpu as pltpu
from jax.experimental.pallas import tpu_sc as plsc

N = 10000
NPAD = 10240
E = 320000
F = 128

NC = 2
NS = 16
NW = NC * NS
CHUNK = 128
NCHUNK = E // CHUNK
CPW = -(-NCHUNK // NW)
ROWS_PER_SUB = NPAD // NS
E2 = 327680
NCHUNK2 = E2 // CHUNK
CPW2 = NCHUNK2 // NW
NBUF = 5
EPW = E2 // NW
DEGBLK = 1024

_mesh = plsc.VectorSubcoreMesh(core_axis_name="c", subcore_axis_name="s")


_cp = pltpu.CompilerParams()
if "needs_layout_passes" in pltpu.CompilerParams.__dataclass_fields__:
    _cp = dataclasses.replace(_cp, needs_layout_passes=False)


@functools.partial(
    pl.kernel,
    mesh=_mesh,
    out_type=jax.ShapeDtypeStruct((NW, NPAD), jnp.float32),
    compiler_params=_cp,
    scratch_types=[
        pltpu.VMEM((DEGBLK,), jnp.int32),
        pltpu.VMEM((NPAD,), jnp.float32),
    ],
)
def _deg_kernel(dst_hbm, zeros_hbm, out_hbm, idx_v, deg_v):
    cid = lax.axis_index("c")
    sid = lax.axis_index("s")
    w = sid * NC + cid
    pltpu.sync_copy(zeros_hbm, deg_v)
    ones = jnp.ones((16,), jnp.float32)
    base = w * EPW

    def scan_block(off, size):
        pltpu.sync_copy(dst_hbm.at[pl.ds(base + off, size)],
                        idx_v.at[pl.ds(0, size)])

        @pl.loop(0, size, step=16)
        def _(j):
            plsc.addupdate_scatter(deg_v, [idx_v[pl.ds(j, 16)]], ones)

    @pl.loop(0, EPW // DEGBLK)
    def _(k):
        scan_block(k * DEGBLK, DEGBLK)

    if EPW % DEGBLK:
        scan_block((EPW // DEGBLK) * DEGBLK, EPW % DEGBLK)

    pltpu.sync_copy(deg_v, out_hbm.at[w])


IDXBLK = 8
HCHUNK = CHUNK // 2
NBLK = CPW2 // IDXBLK


@functools.partial(
    pl.kernel,
    mesh=_mesh,
    out_type=jax.ShapeDtypeStruct((NC, NPAD, F), jnp.float32),
    scratch_types=[
        pltpu.VMEM((IDXBLK, CHUNK), jnp.int32),
        pltpu.VMEM((IDXBLK, CHUNK), jnp.int32),
        pltpu.VMEM((IDXBLK, CHUNK), jnp.int32),
        pltpu.VMEM((IDXBLK, CHUNK), jnp.int32),
        pltpu.VMEM((CHUNK, F), jnp.float32),
        pltpu.VMEM((CHUNK, F), jnp.float32),
        pltpu.VMEM((HCHUNK,), jnp.int32),
        pltpu.VMEM((HCHUNK,), jnp.int32),
        pltpu.VMEM((HCHUNK,), jnp.int32),
        pltpu.VMEM((HCHUNK,), jnp.int32),
        pltpu.VMEM((CHUNK,), jnp.int32),
        pltpu.VMEM((CHUNK,), jnp.int32),
        pltpu.VMEM_SHARED((NPAD, F), jnp.float32),
        pltpu.SemaphoreType.DMA,
        pltpu.SemaphoreType.DMA,
        pltpu.SemaphoreType.DMA,
        pltpu.SemaphoreType.DMA,
    ],
)
def _agg_kernel(y_hbm, src2_hbm, dst2_hbm, out_hbm,
                si_a, si_b, di_a, di_b, rows0, rows1,
                dh0a, dh0b, dh1a, dh1b, sc0, sc1,
                acc_sh, gs0, gs1, ss_a, ss_b):
    cid = lax.axis_index("c")
    sid = lax.axis_index("s")
    rr0 = sid * ROWS_PER_SUB

    @pl.loop(0, CHUNK)
    def _(r):
        for c in range(0, F, 16):
            rows0[r, pl.ds(c, 16)] = jnp.zeros((16,), jnp.float32)

    @pl.loop(0, ROWS_PER_SUB, step=CHUNK)
    def _(rr):
        pltpu.sync_copy(rows0, acc_sh.at[pl.ds(rr0 + rr, CHUNK)])

    w = sid * NC + cid
    row0 = w * CPW2

    def load_blk(b, si, di):
        pltpu.sync_copy(src2_hbm.at[pl.ds(row0 + b * IDXBLK, IDXBLK)], si)
        pltpu.sync_copy(dst2_hbm.at[pl.ds(row0 + b * IDXBLK, IDXBLK)], di)

    def gather(si, m, rbuf, sem, sc):
        for t in range(0, CHUNK, 16):
            sc[pl.ds(t, 16)] = si[m, pl.ds(t, 16)]
        pltpu.async_copy(y_hbm.at[sc], rbuf, sem)

    def wait_g(rbuf, sem):
        pltpu.make_async_copy(y_hbm.at[pl.ds(0, CHUNK)], rbuf, sem).wait()

    def scatter(di, m, rbuf, dca, dcb):
        for t in range(0, HCHUNK, 16):
            dca[pl.ds(t, 16)] = di[m, pl.ds(t, 16)]
            dcb[pl.ds(t, 16)] = di[m, pl.ds(HCHUNK + t, 16)]
        pltpu.async_copy(rbuf.at[pl.ds(0, HCHUNK)], acc_sh.at[dca], ss_a,
                         add=True)
        pltpu.async_copy(rbuf.at[pl.ds(HCHUNK, HCHUNK)], acc_sh.at[dcb], ss_b,
                         add=True)
        pltpu.make_async_copy(y_hbm.at[pl.ds(0, HCHUNK)],
                              rbuf.at[pl.ds(0, HCHUNK)], ss_a).wait()
        pltpu.make_async_copy(y_hbm.at[pl.ds(0, HCHUNK)],
                              rbuf.at[pl.ds(HCHUNK, HCHUNK)], ss_b).wait()

    def process(si, di, nsi, next_guard):
        @pl.loop(0, (IDXBLK - 2) // 2)
        def _(mm):
            m0 = 2 * mm
            wait_g(rows0, gs0)
            scatter(di, m0, rows0, dh0a, dh0b)
            gather(si, m0 + 2, rows0, gs0, sc0)
            wait_g(rows1, gs1)
            scatter(di, m0 + 1, rows1, dh1a, dh1b)
            gather(si, m0 + 3, rows1, gs1, sc1)

        wait_g(rows0, gs0)
        scatter(di, IDXBLK - 2, rows0, dh0a, dh0b)
        if next_guard is None:
            gather(nsi, 0, rows0, gs0, sc0)
        else:
            @pl.when(next_guard)
            def _():
                gather(nsi, 0, rows0, gs0, sc0)
        wait_g(rows1, gs1)
        scatter(di, IDXBLK - 1, rows1, dh1a, dh1b)
        if next_guard is None:
            gather(nsi, 1, rows1, gs1, sc1)
        else:
            @pl.when(next_guard)
            def _():
                gather(nsi, 1, rows1, gs1, sc1)

    plsc.subcore_barrier()

    load_blk(0, si_a, di_a)
    gather(si_a, 0, rows0, gs0, sc0)
    gather(si_a, 1, rows1, gs1, sc1)

    @pl.loop(0, NBLK // 2)
    def _(j):
        b_a = 2 * j
        load_blk(b_a + 1, si_b, di_b)
        process(si_a, di_a, si_b, None)

        @pl.when(j < NBLK // 2 - 1)
        def _():
            load_blk(b_a + 2, si_a, di_a)

        process(si_b, di_b, si_a, j < NBLK // 2 - 1)

    plsc.subcore_barrier()
    pltpu.sync_copy(acc_sh.at[pl.ds(rr0, ROWS_PER_SUB)],
                    out_hbm.at[cid].at[pl.ds(rr0, ROWS_PER_SUB)])


_MM_BLK = 1024


def _mm_body(x_ref, w_ref, b_ref, h_ref):
    h_ref[...] = (
        jnp.dot(x_ref[...], w_ref[...], preferred_element_type=jnp.float32)
        + b_ref[...]
    )


def _matmul(x, W, b2):
    return pl.pallas_call(
        _mm_body,
        grid=(NPAD // _MM_BLK,),
        in_specs=[
            pl.BlockSpec((_MM_BLK, F), lambda i: (i, 0)),
            pl.BlockSpec((F, F), lambda i: (0, 0)),
            pl.BlockSpec((1, F), lambda i: (0, 0)),
        ],
        out_specs=pl.BlockSpec((_MM_BLK, F), lambda i: (i, 0)),
        out_shape=jax.ShapeDtypeStruct((NPAD, F), jnp.float32),
    )(x, W, b2)


def _dis_from_parts(dp):
    deg = jnp.sum(dp, axis=0)[:, None]
    return lax.rsqrt(jnp.maximum(deg, 1.0))


def _scale_body(h_ref, dp_ref, y_ref):
    y_ref[...] = h_ref[...] * _dis_from_parts(dp_ref[...])


def _scale(h, deg_parts):
    return pl.pallas_call(
        _scale_body,
        grid=(NPAD // _MM_BLK,),
        in_specs=[
            pl.BlockSpec((_MM_BLK, F), lambda i: (i, 0)),
            pl.BlockSpec((NW, _MM_BLK), lambda i: (0, i)),
        ],
        out_specs=pl.BlockSpec((_MM_BLK, F), lambda i: (i, 0)),
        out_shape=jax.ShapeDtypeStruct((NPAD, F), jnp.float32),
    )(h, deg_parts)


def _final_body(acc_ref, dp_ref, o_ref):
    z = (acc_ref[0] + acc_ref[1]) * _dis_from_parts(dp_ref[...])
    m = jnp.max(z, axis=1, keepdims=True)
    lse = jnp.log(jnp.sum(jnp.exp(z - m), axis=1, keepdims=True)) + m
    o_ref[...] = z - lse


def _final(acc, deg_parts):
    return pl.pallas_call(
        _final_body,
        grid=(NPAD // _MM_BLK,),
        in_specs=[
            pl.BlockSpec((NC, _MM_BLK, F), lambda i: (0, i, 0)),
            pl.BlockSpec((NW, _MM_BLK), lambda i: (0, i)),
        ],
        out_specs=pl.BlockSpec((_MM_BLK, F), lambda i: (i, 0)),
        out_shape=jax.ShapeDtypeStruct((N, F), jnp.float32),
    )(acc, deg_parts)


def kernel(inputs, edge_index, epoch, W, b):
    del epoch
    ei_flat = edge_index.astype(jnp.int32).reshape(2 * E)
    src = ei_flat[:E]
    dst = ei_flat[E:]
    pad = E2 - E
    pad_iota = jnp.arange(pad, dtype=jnp.int32)
    src_p = jnp.concatenate([src, pad_iota % N])
    dst_p = jnp.concatenate([dst, N + pad_iota % (NPAD - N)])
    src2d = src_p.reshape(NCHUNK2, CHUNK)
    dst2d = dst_p.reshape(NCHUNK2, CHUNK)
    zeros1 = jnp.zeros((NPAD,), jnp.float32)
    b2 = b.reshape(1, F)

    deg_parts = _deg_kernel(dst_p, zeros1)
    h = _matmul(inputs, W, b2)
    y = _scale(h, deg_parts)
    acc = _agg_kernel(y, src2d, dst2d)
    return _final(acc, deg_parts)


# speedup vs baseline: 1.0496x; 1.0073x over previous
import dataclasses
import functools

import jax
import jax.numpy as jnp
from jax import lax
from jax.experimental import pallas as pl
from jax.experimental.pallas import tpu as pltpu
from jax.experimental.pallas import tpu_sc as plsc

N = 10000
NPAD = 10240
E = 320000
F = 128

NC = 2
NS = 16
NW = NC * NS
CHUNK = 128
NCHUNK = E // CHUNK
CPW = -(-NCHUNK // NW)
ROWS_PER_SUB = NPAD // NS
E2 = 327680
NCHUNK2 = E2 // CHUNK
CPW2 = NCHUNK2 // NW
NBUF = 5
EPW = E2 // NW
DEGBLK = 1024

_mesh = plsc.VectorSubcoreMesh(core_axis_name="c", subcore_axis_name="s")


_cp = pltpu.CompilerParams()
if "needs_layout_passes" in pltpu.CompilerParams.__dataclass_fields__:
    _cp = dataclasses.replace(_cp, needs_layout_passes=False)


@functools.partial(
    pl.kernel,
    mesh=_mesh,
    out_type=jax.ShapeDtypeStruct((NW, NPAD), jnp.float32),
    compiler_params=_cp,
    scratch_types=[
        pltpu.VMEM((DEGBLK,), jnp.int32),
        pltpu.VMEM((NPAD,), jnp.float32),
    ],
)
def _deg_kernel(ei1_hbm, zeros_hbm, out_hbm, idx_v, deg_v):
    cid = lax.axis_index("c")
    sid = lax.axis_index("s")
    w = sid * NC + cid
    pltpu.sync_copy(zeros_hbm, deg_v)
    ones = jnp.ones((16,), jnp.float32)
    base = E2 + w * EPW

    @pl.loop(0, EPW // DEGBLK)
    def _(k):
        pltpu.sync_copy(ei1_hbm.at[pl.ds(base + k * DEGBLK, DEGBLK)], idx_v)

        @pl.loop(0, DEGBLK, step=16)
        def _(j):
            plsc.addupdate_scatter(deg_v, [idx_v[pl.ds(j, 16)]], ones)

    pltpu.sync_copy(deg_v, out_hbm.at[w])


IDXBLK = 8
HCHUNK = CHUNK // 2
NBLK = CPW2 // IDXBLK


@functools.partial(
    pl.kernel,
    mesh=_mesh,
    out_type=jax.ShapeDtypeStruct((NC, NPAD, F), jnp.float32),
    scratch_types=[
        pltpu.VMEM((IDXBLK, CHUNK), jnp.int32),
        pltpu.VMEM((IDXBLK, CHUNK), jnp.int32),
        pltpu.VMEM((IDXBLK, CHUNK), jnp.int32),
        pltpu.VMEM((IDXBLK, CHUNK), jnp.int32),
        pltpu.VMEM((CHUNK, F), jnp.float32),
        pltpu.VMEM((CHUNK, F), jnp.float32),
        pltpu.VMEM((HCHUNK,), jnp.int32),
        pltpu.VMEM((HCHUNK,), jnp.int32),
        pltpu.VMEM((HCHUNK,), jnp.int32),
        pltpu.VMEM((HCHUNK,), jnp.int32),
        pltpu.VMEM((CHUNK,), jnp.int32),
        pltpu.VMEM((CHUNK,), jnp.int32),
        pltpu.VMEM_SHARED((NPAD, F), jnp.float32),
        pltpu.SemaphoreType.DMA,
        pltpu.SemaphoreType.DMA,
        pltpu.SemaphoreType.DMA,
        pltpu.SemaphoreType.DMA,
    ],
)
def _agg_kernel(y_hbm, ei2_hbm, out_hbm,
                si_a, si_b, di_a, di_b, rows0, rows1,
                dh0a, dh0b, dh1a, dh1b, sc0, sc1,
                acc_sh, gs0, gs1, ss_a, ss_b):
    cid = lax.axis_index("c")
    sid = lax.axis_index("s")
    rr0 = sid * ROWS_PER_SUB

    @pl.loop(0, CHUNK)
    def _(r):
        for c in range(0, F, 16):
            rows0[r, pl.ds(c, 16)] = jnp.zeros((16,), jnp.float32)

    @pl.loop(0, ROWS_PER_SUB, step=CHUNK)
    def _(rr):
        pltpu.sync_copy(rows0, acc_sh.at[pl.ds(rr0 + rr, CHUNK)])

    w = sid * NC + cid
    row0 = w * CPW2

    def load_blk(b, si, di):
        pltpu.sync_copy(ei2_hbm.at[pl.ds(row0 + b * IDXBLK, IDXBLK)], si)
        pltpu.sync_copy(
            ei2_hbm.at[pl.ds(NCHUNK2 + row0 + b * IDXBLK, IDXBLK)], di)

    def gather(si, m, rbuf, sem, sc):
        for t in range(0, CHUNK, 16):
            sc[pl.ds(t, 16)] = si[m, pl.ds(t, 16)]
        pltpu.async_copy(y_hbm.at[sc], rbuf, sem)

    def wait_g(rbuf, sem):
        pltpu.make_async_copy(y_hbm.at[pl.ds(0, CHUNK)], rbuf, sem).wait()

    def scatter(di, m, rbuf, dca, dcb):
        for t in range(0, HCHUNK, 16):
            dca[pl.ds(t, 16)] = di[m, pl.ds(t, 16)]
            dcb[pl.ds(t, 16)] = di[m, pl.ds(HCHUNK + t, 16)]
        pltpu.async_copy(rbuf.at[pl.ds(0, HCHUNK)], acc_sh.at[dca], ss_a,
                         add=True)
        pltpu.async_copy(rbuf.at[pl.ds(HCHUNK, HCHUNK)], acc_sh.at[dcb], ss_b,
                         add=True)
        pltpu.make_async_copy(y_hbm.at[pl.ds(0, HCHUNK)],
                              rbuf.at[pl.ds(0, HCHUNK)], ss_a).wait()
        pltpu.make_async_copy(y_hbm.at[pl.ds(0, HCHUNK)],
                              rbuf.at[pl.ds(HCHUNK, HCHUNK)], ss_b).wait()

    def process(si, di, nsi, next_guard):
        @pl.loop(0, (IDXBLK - 2) // 2)
        def _(mm):
            m0 = 2 * mm
            wait_g(rows0, gs0)
            scatter(di, m0, rows0, dh0a, dh0b)
            gather(si, m0 + 2, rows0, gs0, sc0)
            wait_g(rows1, gs1)
            scatter(di, m0 + 1, rows1, dh1a, dh1b)
            gather(si, m0 + 3, rows1, gs1, sc1)

        wait_g(rows0, gs0)
        scatter(di, IDXBLK - 2, rows0, dh0a, dh0b)
        if next_guard is None:
            gather(nsi, 0, rows0, gs0, sc0)
        else:
            @pl.when(next_guard)
            def _():
                gather(nsi, 0, rows0, gs0, sc0)
        wait_g(rows1, gs1)
        scatter(di, IDXBLK - 1, rows1, dh1a, dh1b)
        if next_guard is None:
            gather(nsi, 1, rows1, gs1, sc1)
        else:
            @pl.when(next_guard)
            def _():
                gather(nsi, 1, rows1, gs1, sc1)

    plsc.subcore_barrier()

    load_blk(0, si_a, di_a)
    gather(si_a, 0, rows0, gs0, sc0)
    gather(si_a, 1, rows1, gs1, sc1)

    @pl.loop(0, NBLK // 2)
    def _(j):
        b_a = 2 * j
        load_blk(b_a + 1, si_b, di_b)
        process(si_a, di_a, si_b, None)

        @pl.when(j < NBLK // 2 - 1)
        def _():
            load_blk(b_a + 2, si_a, di_a)

        process(si_b, di_b, si_a, j < NBLK // 2 - 1)

    plsc.subcore_barrier()
    pltpu.sync_copy(acc_sh.at[pl.ds(rr0, ROWS_PER_SUB)],
                    out_hbm.at[cid].at[pl.ds(rr0, ROWS_PER_SUB)])


_MM_BLK = 1024


def _mm_body(x_ref, w_ref, b_ref, h_ref):
    h_ref[...] = (
        jnp.dot(x_ref[...], w_ref[...], preferred_element_type=jnp.float32)
        + b_ref[...]
    )


def _matmul(x, W, b2):
    return pl.pallas_call(
        _mm_body,
        grid=(NPAD // _MM_BLK,),
        in_specs=[
            pl.BlockSpec((_MM_BLK, F), lambda i: (i, 0)),
            pl.BlockSpec((F, F), lambda i: (0, 0)),
            pl.BlockSpec((1, F), lambda i: (0, 0)),
        ],
        out_specs=pl.BlockSpec((_MM_BLK, F), lambda i: (i, 0)),
        out_shape=jax.ShapeDtypeStruct((NPAD, F), jnp.float32),
    )(x, W, b2)


def _dis_from_parts(dp):
    deg = jnp.sum(dp, axis=0)[:, None]
    return lax.rsqrt(jnp.maximum(deg, 1.0))


def _scale_body(h_ref, dp_ref, y_ref):
    y_ref[...] = h_ref[...] * _dis_from_parts(dp_ref[...])


def _scale(h, deg_parts):
    return pl.pallas_call(
        _scale_body,
        grid=(NPAD // _MM_BLK,),
        in_specs=[
            pl.BlockSpec((_MM_BLK, F), lambda i: (i, 0)),
            pl.BlockSpec((NW, _MM_BLK), lambda i: (0, i)),
        ],
        out_specs=pl.BlockSpec((_MM_BLK, F), lambda i: (i, 0)),
        out_shape=jax.ShapeDtypeStruct((NPAD, F), jnp.float32),
    )(h, deg_parts)


def _final_body(acc_ref, dp_ref, o_ref):
    z = (acc_ref[0] + acc_ref[1]) * _dis_from_parts(dp_ref[...])
    m = jnp.max(z, axis=1, keepdims=True)
    lse = jnp.log(jnp.sum(jnp.exp(z - m), axis=1, keepdims=True)) + m
    o_ref[...] = z - lse


def _final(acc, deg_parts):
    return pl.pallas_call(
        _final_body,
        grid=(NPAD // _MM_BLK,),
        in_specs=[
            pl.BlockSpec((NC, _MM_BLK, F), lambda i: (0, i, 0)),
            pl.BlockSpec((NW, _MM_BLK), lambda i: (0, i)),
        ],
        out_specs=pl.BlockSpec((_MM_BLK, F), lambda i: (i, 0)),
        out_shape=jax.ShapeDtypeStruct((N, F), jnp.float32),
    )(acc, deg_parts)


def kernel(inputs, edge_index, epoch, W, b):
    del epoch
    pad = E2 - E
    pad_iota = jnp.arange(pad, dtype=jnp.int32)
    pad2 = jnp.stack([pad_iota % N, N + pad_iota % (NPAD - N)])
    ei2 = jnp.concatenate([edge_index.astype(jnp.int32), pad2],
                          axis=1).reshape(2 * NCHUNK2, CHUNK)
    zeros1 = jnp.zeros((NPAD,), jnp.float32)
    b2 = b.reshape(1, F)

    deg_parts = _deg_kernel(ei2.reshape(2 * E2), zeros1)
    h = _matmul(inputs, W, b2)
    y = _scale(h, deg_parts)
    acc = _agg_kernel(y, ei2)
    return _final(acc, deg_parts)
